# baseline (speedup 1.0000x reference)
import functools

import jax
import jax.numpy as jnp
from jax import lax
from jax.experimental import pallas as pl
from jax.experimental.pallas import tpu as pltpu
from jax.experimental.pallas import tpu_sc as plsc

NUM_EMB = 1_000_000
DIM = 64

NC = 2
NS = 16
NW = NC * NS

B_TOK, T_TOK = 16384, 50
BAND = B_TOK // NW
BW = 128
J = BAND // BW
N_BLK = T_TOK * J
DT, DR = DIM // 8, 8
PITCH = BW + 1
NBUF = 3

WPAN = 128
N_PAN = NUM_EMB // WPAN
PAN_REM = NUM_EMB - N_PAN * WPAN

_mesh = plsc.VectorSubcoreMesh(
    core_axis_name="c", subcore_axis_name="s", num_cores=NC, num_subcores=NS
)


@functools.partial(
    pl.kernel,
    out_type=jax.ShapeDtypeStruct((NUM_EMB // 2, 2 * DIM), jnp.float32),
    mesh=_mesh,
    scratch_types=[
        pltpu.VMEM((2, DIM, WPAN), jnp.float32),
        pltpu.VMEM((2, WPAN // 2, 2 * DIM), jnp.float32),
        pltpu.SemaphoreType.DMA((2,)),
        pltpu.SemaphoreType.DMA((2,)),
    ],
    compiler_params=pltpu.CompilerParams(
        use_tc_tiling_on_sc=True, needs_layout_passes=False
    ),
)
def _transpose_weight(wt_hbm, tail_hbm, tbl_hbm, blk_v, tr_v, sem_i, sem_o):
    wid = lax.axis_index("s") * NC + lax.axis_index("c")

    iota = lax.iota(jnp.int32, 16)
    m16 = [(iota + k) % 16 for k in range(16)]
    pack = [(iota % 2) * DIM + m16[k] for k in range(16)]
    half = iota // 2

    def fire_in(c, slot):
        pltpu.async_copy(
            wt_hbm.at[:, pl.ds(c * WPAN, WPAN)], blk_v.at[slot], sem_i.at[slot]
        )

    def drain_in(slot):
        pltpu.make_async_copy(
            wt_hbm.at[:, pl.ds(0, WPAN)], blk_v.at[slot], sem_i.at[slot]
        ).wait()

    def transpose(slot):
        blk = blk_v.at[slot]
        tr = tr_v.at[slot]
        for d0 in range(0, DIM, 16):

            @pl.loop(0, WPAN, step=16)
            def _i(i0):
                i_vec = i0 + iota
                r_vec = lax.div(i0, 2) + half
                for k in range(16):
                    vec = plsc.load_gather(blk, [d0 + m16[k], i_vec])
                    plsc.store_scatter(tr, [r_vec, d0 + pack[k]], vec)

    def fire_out(c, slot):
        pltpu.async_copy(
            tr_v.at[slot],
            tbl_hbm.at[pl.ds(c * (WPAN // 2), WPAN // 2), :],
            sem_o.at[slot],
        )

    def wait_out(slot):
        pltpu.make_async_copy(
            tr_v.at[slot], tbl_hbm.at[pl.ds(0, WPAN // 2), :], sem_o.at[slot]
        ).wait()

    n_mine = N_PAN // NW
    extra = N_PAN - n_mine * NW

    fire_in(wid, 0)
    fire_in(wid + NW, 1)

    @pl.loop(0, n_mine, step=2)
    def _pan(p):
        for s in range(2):
            idx = p + s
            c = wid + idx * NW
            drain_in(s)

            @pl.when(idx >= 2)
            def _():
                wait_out(s)

            transpose(s)
            fire_out(c, s)

            @pl.when(idx < n_mine - 2)
            def _():
                fire_in(c + 2 * NW, s)

    wait_out(0)

    @pl.when(wid < extra)
    def _():
        c = n_mine * NW + wid
        fire_in(c, 0)
        drain_in(0)
        transpose(0)
        fire_out(c, 0)
        wait_out(0)

    @pl.when(wid == extra)
    def _():
        pltpu.async_copy(
            tail_hbm,
            tbl_hbm.at[pl.ds(N_PAN * (WPAN // 2), PAN_REM // 2), :],
            sem_i.at[0],
        )
        pltpu.make_async_copy(
            tail_hbm, tbl_hbm.at[pl.ds(0, PAN_REM // 2), :], sem_i.at[0]
        ).wait()

    wait_out(1)


@functools.partial(
    pl.kernel,
    out_type=jax.ShapeDtypeStruct((T_TOK, DT, B_TOK // BW, DR, BW), jnp.float32),
    mesh=_mesh,
    scratch_types=[
        pltpu.VMEM((T_TOK, BAND), jnp.int32),
        pltpu.VMEM((NBUF, BW, DIM), jnp.float32),
        pltpu.VMEM((NBUF, DT, DR, PITCH), jnp.float32),
        pltpu.SemaphoreType.DMA((NBUF,)),
        pltpu.SemaphoreType.DMA((NBUF,)),
    ],
    compiler_params=pltpu.CompilerParams(
        use_tc_tiling_on_sc=False, needs_layout_passes=False
    ),
)
def _emb_gather(table_hbm, idx_hbm, out_hbm, idx_v, rows_v, tr_v, sem_g, sem_o):
    wid = lax.axis_index("s") * NC + lax.axis_index("c")
    bt_base = wid * J

    pltpu.sync_copy(idx_hbm.at[:, wid, :], idx_v)

    iota = lax.iota(jnp.int32, 16)
    dt_idx = [(h * 16 + iota) // 8 for h in range(DIM // 16)]
    dr_idx = iota % 8

    def fire_g(i, slot):
        t = lax.shift_right_logical(i, 2)
        j = lax.bitwise_and(i, 3)
        pltpu.async_copy(
            table_hbm.at[idx_v.at[t, pl.ds(j * BW, BW)]],
            rows_v.at[slot],
            sem_g.at[slot],
        )

    def drain_g(slot):
        pltpu.make_async_copy(
            table_hbm.at[pl.ds(0, BW)], rows_v.at[slot], sem_g.at[slot]
        ).wait()

    def permute(slot):
        rows = rows_v.at[slot]
        tr = tr_v.at[slot]

        @pl.loop(0, BW, unroll=2)
        def _b(b):
            b_vec = jnp.full((16,), b, jnp.int32)
            for h in range(DIM // 16):
                vec = rows[b, pl.ds(h * 16, 16)]
                plsc.store_scatter(tr, [dt_idx[h], dr_idx, b_vec], vec)

    def fire_w(i, slot):
        t = lax.shift_right_logical(i, 2)
        j = lax.bitwise_and(i, 3)
        pltpu.async_copy(
            tr_v.at[slot].at[:, :, pl.ds(0, BW)],
            out_hbm.at[t, :, bt_base + j, :, :],
            sem_o.at[slot],
        )

    def wait_w(slot):
        pltpu.make_async_copy(
            tr_v.at[slot].at[:, :, pl.ds(0, BW)],
            out_hbm.at[0, :, 0, :, :],
            sem_o.at[slot],
        ).wait()

    def body(i, slot, first, last):
        drain_g(slot)
        if not first:
            wait_w(slot)
        permute(slot)
        fire_w(i, slot)
        if not last:
            fire_g(i + NBUF, slot)

    for i in range(NBUF):
        fire_g(i, i)
    for i in range(NBUF):
        body(i, i, True, False)

    @pl.loop(NBUF, N_BLK - 5, step=NBUF)
    def _grp(i0):
        for d in range(NBUF):
            body(i0 + d, d, False, False)

    for i in range(N_BLK - 5, N_BLK):
        body(i, i % NBUF, False, i + NBUF >= N_BLK)
    for i in range(N_BLK - NBUF, N_BLK):
        wait_w(i % NBUF)


def kernel(token_ids, weight):
    tail = weight[N_PAN * WPAN :, :].reshape(PAN_REM // 2, 2 * DIM)
    tbl_wide = _transpose_weight(weight.T, tail)
    tbl = tbl_wide.reshape(NUM_EMB, DIM)
    idx3d = token_ids.T.reshape(T_TOK, NW, BAND).astype(jnp.int32)
    out5d = _emb_gather(tbl, idx3d)
    out = jnp.transpose(out5d, (2, 4, 0, 1, 3)).reshape(B_TOK, T_TOK, DIM)
    return out


# speedup vs baseline: 1.0056x; 1.0056x over previous
import functools

import jax
import jax.numpy as jnp
from jax import lax
from jax.experimental import pallas as pl
from jax.experimental.pallas import tpu as pltpu
from jax.experimental.pallas import tpu_sc as plsc

NUM_EMB = 1_000_000
DIM = 64

NC = 2
NS = 16
NW = NC * NS

B_TOK, T_TOK = 16384, 50
BAND = B_TOK // NW
BW = 128
J = BAND // BW
N_BLK = T_TOK * J
DT, DR = DIM // 8, 8
PITCH = BW + 1
NBUF = 3

WPAN = 128
N_PAN = NUM_EMB // WPAN
PAN_REM = NUM_EMB - N_PAN * WPAN

_mesh = plsc.VectorSubcoreMesh(
    core_axis_name="c", subcore_axis_name="s", num_cores=NC, num_subcores=NS
)


@functools.partial(
    pl.kernel,
    out_type=jax.ShapeDtypeStruct((NUM_EMB // 2, 2 * DIM), jnp.float32),
    mesh=_mesh,
    scratch_types=[
        pltpu.VMEM((2, DIM, WPAN), jnp.float32),
        pltpu.VMEM((2, WPAN // 2, 2 * DIM), jnp.float32),
        pltpu.SemaphoreType.DMA((2,)),
        pltpu.SemaphoreType.DMA((2,)),
    ],
    compiler_params=pltpu.CompilerParams(
        use_tc_tiling_on_sc=True, needs_layout_passes=False
    ),
)
def _transpose_weight(wt_hbm, tail_hbm, tbl_hbm, blk_v, tr_v, sem_i, sem_o):
    wid = lax.axis_index("s") * NC + lax.axis_index("c")

    iota = lax.iota(jnp.int32, 16)
    m16 = [(iota + k) % 16 for k in range(16)]
    pack = [(iota % 2) * DIM + m16[k] for k in range(16)]
    half = iota // 2

    def fire_in(c, slot):
        pltpu.async_copy(
            wt_hbm.at[:, pl.ds(c * WPAN, WPAN)], blk_v.at[slot], sem_i.at[slot]
        )

    def drain_in(slot):
        pltpu.make_async_copy(
            wt_hbm.at[:, pl.ds(0, WPAN)], blk_v.at[slot], sem_i.at[slot]
        ).wait()

    def transpose(slot):
        blk = blk_v.at[slot]
        tr = tr_v.at[slot]
        for d0 in range(0, DIM, 16):
            d_vecs = [d0 + m16[k] for k in range(16)]
            c_vecs = [d0 + pack[k] for k in range(16)]

            @pl.loop(0, WPAN, step=16)
            def _i(i0):
                i_vec = i0 + iota
                r_vec = lax.div(i0, 2) + half
                for k in range(16):
                    vec = plsc.load_gather(blk, [d_vecs[k], i_vec])
                    plsc.store_scatter(tr, [r_vec, c_vecs[k]], vec)

    def fire_out(c, slot):
        pltpu.async_copy(
            tr_v.at[slot],
            tbl_hbm.at[pl.ds(c * (WPAN // 2), WPAN // 2), :],
            sem_o.at[slot],
        )

    def wait_out(slot):
        pltpu.make_async_copy(
            tr_v.at[slot], tbl_hbm.at[pl.ds(0, WPAN // 2), :], sem_o.at[slot]
        ).wait()

    n_mine = N_PAN // NW
    extra = N_PAN - n_mine * NW

    fire_in(wid, 0)
    fire_in(wid + NW, 1)

    @pl.loop(0, n_mine, step=2)
    def _pan(p):
        for s in range(2):
            idx = p + s
            c = wid + idx * NW
            drain_in(s)

            @pl.when(idx >= 2)
            def _():
                wait_out(s)

            transpose(s)
            fire_out(c, s)

            @pl.when(idx < n_mine - 2)
            def _():
                fire_in(c + 2 * NW, s)

    wait_out(0)

    @pl.when(wid < extra)
    def _():
        c = n_mine * NW + wid
        fire_in(c, 0)
        drain_in(0)
        transpose(0)
        fire_out(c, 0)
        wait_out(0)

    @pl.when(wid == extra)
    def _():
        pltpu.async_copy(
            tail_hbm,
            tbl_hbm.at[pl.ds(N_PAN * (WPAN // 2), PAN_REM // 2), :],
            sem_i.at[0],
        )
        pltpu.make_async_copy(
            tail_hbm, tbl_hbm.at[pl.ds(0, PAN_REM // 2), :], sem_i.at[0]
        ).wait()

    wait_out(1)


@functools.partial(
    pl.kernel,
    out_type=jax.ShapeDtypeStruct((T_TOK, DT, B_TOK // BW, DR, BW), jnp.float32),
    mesh=_mesh,
    scratch_types=[
        pltpu.VMEM((T_TOK, BAND), jnp.int32),
        pltpu.VMEM((NBUF, BW, DIM), jnp.float32),
        pltpu.VMEM((NBUF, DT, DR, PITCH), jnp.float32),
        pltpu.SemaphoreType.DMA((NBUF,)),
        pltpu.SemaphoreType.DMA((NBUF,)),
    ],
    compiler_params=pltpu.CompilerParams(
        use_tc_tiling_on_sc=False, needs_layout_passes=False
    ),
)
def _emb_gather(table_hbm, idx_hbm, out_hbm, idx_v, rows_v, tr_v, sem_g, sem_o):
    wid = lax.axis_index("s") * NC + lax.axis_index("c")
    bt_base = wid * J

    pltpu.sync_copy(idx_hbm.at[:, wid, :], idx_v)

    iota = lax.iota(jnp.int32, 16)
    dt_idx = [(h * 16 + iota) // 8 for h in range(DIM // 16)]
    dr_idx = iota % 8

    def fire_g(i, slot):
        t = lax.shift_right_logical(i, 2)
        j = lax.bitwise_and(i, 3)
        pltpu.async_copy(
            table_hbm.at[idx_v.at[t, pl.ds(j * BW, BW)]],
            rows_v.at[slot],
            sem_g.at[slot],
        )

    def drain_g(slot):
        pltpu.make_async_copy(
            table_hbm.at[pl.ds(0, BW)], rows_v.at[slot], sem_g.at[slot]
        ).wait()

    def permute(slot):
        rows = rows_v.at[slot]
        tr = tr_v.at[slot]

        @pl.loop(0, BW, unroll=4)
        def _b(b):
            b_vec = jnp.full((16,), b, jnp.int32)
            for h in range(DIM // 16):
                vec = rows[b, pl.ds(h * 16, 16)]
                plsc.store_scatter(tr, [dt_idx[h], dr_idx, b_vec], vec)

    def fire_w(i, slot):
        t = lax.shift_right_logical(i, 2)
        j = lax.bitwise_and(i, 3)
        pltpu.async_copy(
            tr_v.at[slot].at[:, :, pl.ds(0, BW)],
            out_hbm.at[t, :, bt_base + j, :, :],
            sem_o.at[slot],
        )

    def wait_w(slot):
        pltpu.make_async_copy(
            tr_v.at[slot].at[:, :, pl.ds(0, BW)],
            out_hbm.at[0, :, 0, :, :],
            sem_o.at[slot],
        ).wait()

    def body(i, slot, first, last):
        drain_g(slot)
        if not first:
            wait_w(slot)
        permute(slot)
        fire_w(i, slot)
        if not last:
            fire_g(i + NBUF, slot)

    for i in range(NBUF):
        fire_g(i, i)
    for i in range(NBUF):
        body(i, i, True, False)

    @pl.loop(NBUF, N_BLK - 5, step=NBUF)
    def _grp(i0):
        for d in range(NBUF):
            body(i0 + d, d, False, False)

    for i in range(N_BLK - 5, N_BLK):
        body(i, i % NBUF, False, i + NBUF >= N_BLK)
    for i in range(N_BLK - NBUF, N_BLK):
        wait_w(i % NBUF)


def kernel(token_ids, weight):
    tail = weight[N_PAN * WPAN :, :].reshape(PAN_REM // 2, 2 * DIM)
    tbl_wide = _transpose_weight(weight.T, tail)
    tbl = tbl_wide.reshape(NUM_EMB, DIM)
    idx3d = token_ids.T.reshape(T_TOK, NW, BAND).astype(jnp.int32)
    out5d = _emb_gather(tbl, idx3d)
    out = jnp.transpose(out5d, (2, 4, 0, 1, 3)).reshape(B_TOK, T_TOK, DIM)
    return out


# speedup vs baseline: 1.0245x; 1.0188x over previous
import functools

import jax
import jax.numpy as jnp
from jax import lax
from jax.experimental import pallas as pl
from jax.experimental.pallas import tpu as pltpu
from jax.experimental.pallas import tpu_sc as plsc

NUM_EMB = 1_000_000
DIM = 64

NC = 2
NS = 16
NW = NC * NS

B_TOK, T_TOK = 16384, 50
BAND = B_TOK // NW
BW = 128
J = BAND // BW
N_BLK = T_TOK * J
DT, DR = DIM // 8, 8
PITCH = BW + 1
NBUF = 3

WPAN = 128
N_PAN = NUM_EMB // WPAN
PAN_REM = NUM_EMB - N_PAN * WPAN

_mesh = plsc.VectorSubcoreMesh(
    core_axis_name="c", subcore_axis_name="s", num_cores=NC, num_subcores=NS
)


@functools.partial(
    pl.kernel,
    out_type=jax.ShapeDtypeStruct((NUM_EMB, 2 * DIM), jnp.float32),
    mesh=_mesh,
    scratch_types=[
        pltpu.VMEM((2, DIM, WPAN), jnp.float32),
        pltpu.VMEM((2, WPAN, 2 * DIM), jnp.float32),
        pltpu.SemaphoreType.DMA((2,)),
        pltpu.SemaphoreType.DMA((2,)),
    ],
    compiler_params=pltpu.CompilerParams(
        use_tc_tiling_on_sc=True, needs_layout_passes=False
    ),
)
def _transpose_weight(wt_hbm, tail_hbm, tbl_hbm, blk_v, tr_v, sem_i, sem_o):
    wid = lax.axis_index("s") * NC + lax.axis_index("c")

    iota = lax.iota(jnp.int32, 16)
    m16 = [(iota + k) % 16 for k in range(16)]

    def fire_in(c, slot):
        pltpu.async_copy(
            wt_hbm.at[:, pl.ds(c * WPAN, WPAN)], blk_v.at[slot], sem_i.at[slot]
        )

    def drain_in(slot):
        pltpu.make_async_copy(
            wt_hbm.at[:, pl.ds(0, WPAN)], blk_v.at[slot], sem_i.at[slot]
        ).wait()

    def transpose(slot, width=WPAN):
        blk = blk_v.at[slot]
        tr = tr_v.at[slot]
        for d0 in range(0, DIM, 16):

            @pl.loop(0, width, step=16)
            def _i(i0):
                i_vec = i0 + iota
                for k in range(16):
                    d_vec = d0 + m16[k]
                    vec = plsc.load_gather(blk, [d_vec, i_vec])
                    plsc.store_scatter(tr, [i_vec, d_vec], vec)

    def fire_out(c, slot):
        pltpu.async_copy(
            tr_v.at[slot], tbl_hbm.at[pl.ds(c * WPAN, WPAN), :], sem_o.at[slot]
        )

    def wait_out(slot):
        pltpu.make_async_copy(
            tr_v.at[slot], tbl_hbm.at[pl.ds(0, WPAN), :], sem_o.at[slot]
        ).wait()

    n_mine_min = N_PAN // NW

    fire_in(wid, 0)
    fire_in(wid + NW, 1)
    for s in range(2):
        drain_in(s)
        transpose(s)
        fire_out(wid + s * NW, s)
        fire_in(wid + (s + 2) * NW, s)

    @pl.loop(2, n_mine_min - 2, step=2)
    def _pan(p):
        c = wid + p * NW
        for s in range(2):
            drain_in(s)
            wait_out(s)
            transpose(s)
            fire_out(c + s * NW, s)
            fire_in(c + (s + 2) * NW, s)

    for s, p in ((0, n_mine_min - 2), (1, n_mine_min - 1)):
        c = wid + p * NW
        drain_in(s)
        wait_out(s)
        transpose(s)
        fire_out(c, s)

    extra = N_PAN - n_mine_min * NW

    @pl.when(wid < extra)
    def _():
        c = n_mine_min * NW + wid
        fire_in(c, 0)
        drain_in(0)
        wait_out(0)
        transpose(0)
        fire_out(c, 0)
        wait_out(0)

    @pl.when(wid == extra)
    def _():
        pltpu.async_copy(
            tail_hbm,
            tbl_hbm.at[pl.ds(N_PAN * WPAN, PAN_REM), :],
            sem_i.at[0],
        )
        pltpu.make_async_copy(
            tail_hbm, tbl_hbm.at[pl.ds(0, PAN_REM), :], sem_i.at[0]
        ).wait()

    @pl.when(wid >= extra)
    def _():
        wait_out(0)

    wait_out(1)


@functools.partial(
    pl.kernel,
    out_type=jax.ShapeDtypeStruct((T_TOK, DT, B_TOK // BW, DR, BW), jnp.float32),
    mesh=_mesh,
    scratch_types=[
        pltpu.VMEM((T_TOK, BAND), jnp.int32),
        pltpu.VMEM((NBUF, BW, DIM), jnp.float32),
        pltpu.VMEM((NBUF, DT, DR, PITCH), jnp.float32),
        pltpu.SemaphoreType.DMA((NBUF,)),
        pltpu.SemaphoreType.DMA((NBUF,)),
    ],
    compiler_params=pltpu.CompilerParams(
        use_tc_tiling_on_sc=False, needs_layout_passes=False
    ),
)
def _emb_gather(table_hbm, idx_hbm, out_hbm, idx_v, rows_v, tr_v, sem_g, sem_o):
    wid = lax.axis_index("s") * NC + lax.axis_index("c")
    bt_base = wid * J

    pltpu.sync_copy(idx_hbm.at[:, wid, :], idx_v)

    iota = lax.iota(jnp.int32, 16)
    dt_idx = [(h * 16 + iota) // 8 for h in range(DIM // 16)]
    dr_idx = iota % 8

    def fire_g(i, slot):
        t = lax.shift_right_logical(i, 2)
        j = lax.bitwise_and(i, 3)
        pltpu.async_copy(
            table_hbm.at[idx_v.at[t, pl.ds(j * BW, BW)]],
            rows_v.at[slot],
            sem_g.at[slot],
        )

    def drain_g(slot):
        pltpu.make_async_copy(
            table_hbm.at[pl.ds(0, BW)], rows_v.at[slot], sem_g.at[slot]
        ).wait()

    def permute(slot):
        rows = rows_v.at[slot]
        tr = tr_v.at[slot]

        @pl.loop(0, BW, unroll=4)
        def _b(b):
            b_vec = jnp.full((16,), b, jnp.int32)
            for h in range(DIM // 16):
                vec = rows[b, pl.ds(h * 16, 16)]
                plsc.store_scatter(tr, [dt_idx[h], dr_idx, b_vec], vec)

    def fire_w(i, slot):
        t = lax.shift_right_logical(i, 2)
        j = lax.bitwise_and(i, 3)
        pltpu.async_copy(
            tr_v.at[slot].at[:, :, pl.ds(0, BW)],
            out_hbm.at[t, :, bt_base + j, :, :],
            sem_o.at[slot],
        )

    def wait_w(slot):
        pltpu.make_async_copy(
            tr_v.at[slot].at[:, :, pl.ds(0, BW)],
            out_hbm.at[0, :, 0, :, :],
            sem_o.at[slot],
        ).wait()

    def body(i, slot, first, last):
        drain_g(slot)
        if not first:
            wait_w(slot)
        permute(slot)
        fire_w(i, slot)
        if not last:
            fire_g(i + NBUF, slot)

    for i in range(NBUF):
        fire_g(i, i)
    for i in range(NBUF):
        body(i, i, True, False)

    @pl.loop(NBUF, N_BLK - 5, step=NBUF)
    def _grp(i0):
        for d in range(NBUF):
            body(i0 + d, d, False, False)

    for i in range(N_BLK - 5, N_BLK):
        body(i, i % NBUF, False, i + NBUF >= N_BLK)
    for i in range(N_BLK - NBUF, N_BLK):
        wait_w(i % NBUF)


def kernel(token_ids, weight):
    tail = jnp.pad(weight[N_PAN * WPAN :, :], ((0, 0), (0, DIM)))
    tbl_wide = _transpose_weight(weight.T, tail)
    tbl = tbl_wide.reshape(2 * NUM_EMB, DIM)
    idx3d = (token_ids.T * 2).reshape(T_TOK, NW, BAND).astype(jnp.int32)
    out5d = _emb_gather(tbl, idx3d)
    out = jnp.transpose(out5d, (2, 4, 0, 1, 3)).reshape(B_TOK, T_TOK, DIM)
    return out
